# speedup vs baseline: 1.0413x; 1.0413x over previous
import functools

import jax
import jax.numpy as jnp
from jax import lax
from jax.experimental import pallas as pl
from jax.experimental.pallas import tpu as pltpu
from jax.experimental.pallas import tpu_sc as plsc

E = 320000
NA = 10000
MAX_NB = 32
BOND_FDIM = 144
ATOM_FDIM = 128
H = 128
DEPTH = 5
N_MOLS = 200
MOL_SIZE = 50

NC = 2
NS = 16
NW = NC * NS

A_BLK = 4
IDX_BLK = A_BLK * MAX_NB
GS_NBLK = NA // A_BLK
GS_KMAX = (GS_NBLK + NW - 1) // NW


def _sc_gather_sum_body(msg_hbm, a2b_hbm, out_hbm, idx_v, rows_v, out_v, sem):
    wid = lax.axis_index("s") * NC + lax.axis_index("c")

    def step(k, _):
        blk = k * NW + wid

        @pl.when(blk < GS_NBLK)
        def _():
            pltpu.sync_copy(a2b_hbm.at[pl.ds(blk * IDX_BLK, IDX_BLK)], idx_v)
            pltpu.async_copy(msg_hbm.at[idx_v], rows_v, sem).wait()
            for a in range(A_BLK):
                for c in range(H // 16):
                    acc = rows_v[a * MAX_NB, pl.ds(c * 16, 16)]
                    for r in range(1, MAX_NB):
                        acc = acc + rows_v[a * MAX_NB + r, pl.ds(c * 16, 16)]
                    out_v[a, pl.ds(c * 16, 16)] = acc
            pltpu.sync_copy(out_v, out_hbm.at[pl.ds(blk * A_BLK, A_BLK)])

        return _

    lax.fori_loop(0, GS_KMAX, step, None)


_sc_gather_sum = functools.partial(
    pl.kernel,
    out_type=jax.ShapeDtypeStruct((NA, H), jnp.float32),
    mesh=plsc.VectorSubcoreMesh(core_axis_name="c", subcore_axis_name="s"),
    scratch_types=[
        pltpu.VMEM((IDX_BLK,), jnp.int32),
        pltpu.VMEM((IDX_BLK, H), jnp.float32),
        pltpu.VMEM((A_BLK, H), jnp.float32),
        pltpu.SemaphoreType.DMA,
    ],
)(_sc_gather_sum_body)


B_BLK = 128
DF_NBLK = E // B_BLK
DF_KMAX = (DF_NBLK + NW - 1) // NW


def _sc_diff_body(asum_hbm, msg_hbm, b2a_hbm, b2revb_hbm, out_hbm,
                  ia_v, ir_v, ga_v, gm_v, sem_a, sem_m):
    wid = lax.axis_index("s") * NC + lax.axis_index("c")

    def step(k, _):
        blk = k * NW + wid

        @pl.when(blk < DF_NBLK)
        def _():
            base = blk * B_BLK
            pltpu.sync_copy(b2a_hbm.at[pl.ds(base, B_BLK)], ia_v)
            pltpu.sync_copy(b2revb_hbm.at[pl.ds(base, B_BLK)], ir_v)
            da = pltpu.async_copy(asum_hbm.at[ia_v], ga_v, sem_a)
            dm = pltpu.async_copy(msg_hbm.at[ir_v], gm_v, sem_m)
            da.wait()
            dm.wait()
            for r in range(B_BLK):
                for c in range(H // 16):
                    ga_v[r, pl.ds(c * 16, 16)] = (
                        ga_v[r, pl.ds(c * 16, 16)] - gm_v[r, pl.ds(c * 16, 16)]
                    )
            pltpu.sync_copy(ga_v, out_hbm.at[pl.ds(base, B_BLK)])

        return _

    lax.fori_loop(0, DF_KMAX, step, None)


_sc_diff = functools.partial(
    pl.kernel,
    out_type=jax.ShapeDtypeStruct((E, H), jnp.float32),
    mesh=plsc.VectorSubcoreMesh(core_axis_name="c", subcore_axis_name="s"),
    scratch_types=[
        pltpu.VMEM((B_BLK,), jnp.int32),
        pltpu.VMEM((B_BLK,), jnp.int32),
        pltpu.VMEM((B_BLK, H), jnp.float32),
        pltpu.VMEM((B_BLK, H), jnp.float32),
        pltpu.SemaphoreType.DMA,
        pltpu.SemaphoreType.DMA,
    ],
)(_sc_diff_body)


TC_R = 6400


def _tc_input_body(fb_ref, wi_ref, inp_ref, msg_ref):
    x = jnp.dot(fb_ref[...], wi_ref[...], preferred_element_type=jnp.float32)
    inp_ref[...] = x
    msg_ref[...] = jnp.maximum(x, 0.0)


def _tc_input(f_bonds, W_i):
    grid = (E // TC_R,)
    return pl.pallas_call(
        _tc_input_body,
        grid=grid,
        in_specs=[
            pl.BlockSpec((TC_R, BOND_FDIM), lambda i: (i, 0)),
            pl.BlockSpec((BOND_FDIM, H), lambda i: (0, 0)),
        ],
        out_specs=[
            pl.BlockSpec((TC_R, H), lambda i: (i, 0)),
            pl.BlockSpec((TC_R, H), lambda i: (i, 0)),
        ],
        out_shape=[
            jax.ShapeDtypeStruct((E, H), jnp.float32),
            jax.ShapeDtypeStruct((E, H), jnp.float32),
        ],
    )(f_bonds, W_i)


def _tc_update_body(d_ref, inp_ref, wh_ref, out_ref):
    x = jnp.dot(d_ref[...], wh_ref[...], preferred_element_type=jnp.float32)
    out_ref[...] = jnp.maximum(inp_ref[...] + x, 0.0)


def _tc_update(dmat, inp, W_h):
    grid = (E // TC_R,)
    return pl.pallas_call(
        _tc_update_body,
        grid=grid,
        in_specs=[
            pl.BlockSpec((TC_R, H), lambda i: (i, 0)),
            pl.BlockSpec((TC_R, H), lambda i: (i, 0)),
            pl.BlockSpec((H, H), lambda i: (0, 0)),
        ],
        out_specs=pl.BlockSpec((TC_R, H), lambda i: (i, 0)),
        out_shape=jax.ShapeDtypeStruct((E, H), jnp.float32),
    )(dmat, inp, W_h)


FIN_R = 2000
FIN_M = FIN_R // MOL_SIZE


def _tc_final_body(fa_ref, as_ref, woa_ref, woh_ref, bo_ref, out_ref):
    ah = jnp.dot(fa_ref[...], woa_ref[...], preferred_element_type=jnp.float32)
    ah = ah + jnp.dot(as_ref[...], woh_ref[...],
                      preferred_element_type=jnp.float32)
    ah = jnp.maximum(ah + bo_ref[...], 0.0)
    rows = lax.broadcasted_iota(jnp.int32, (FIN_M, FIN_R), 0)
    cols = lax.broadcasted_iota(jnp.int32, (FIN_M, FIN_R), 1)
    pool = jnp.where(cols // MOL_SIZE == rows, 1.0 / MOL_SIZE, 0.0)
    out_ref[...] = jnp.dot(pool, ah, preferred_element_type=jnp.float32)


def _tc_final(f_atoms, a_sum, Wo_a, Wo_h, b_o2):
    grid = (NA // FIN_R,)
    return pl.pallas_call(
        _tc_final_body,
        grid=grid,
        in_specs=[
            pl.BlockSpec((FIN_R, ATOM_FDIM), lambda i: (i, 0)),
            pl.BlockSpec((FIN_R, H), lambda i: (i, 0)),
            pl.BlockSpec((ATOM_FDIM, H), lambda i: (0, 0)),
            pl.BlockSpec((H, H), lambda i: (0, 0)),
            pl.BlockSpec((1, H), lambda i: (0, 0)),
        ],
        out_specs=pl.BlockSpec((FIN_M, H), lambda i: (i, 0)),
        out_shape=jax.ShapeDtypeStruct((N_MOLS, H), jnp.float32),
    )(f_atoms, a_sum, Wo_a, Wo_h, b_o2)


def kernel(f_atoms, f_bonds, a2b, b2a, b2revb, W_i, W_h, W_o, b_o):
    a2b_flat = a2b.reshape(NA * MAX_NB).astype(jnp.int32)
    b2a = b2a.astype(jnp.int32)
    b2revb = b2revb.astype(jnp.int32)

    inp, msg = _tc_input(f_bonds, W_i)
    for _ in range(DEPTH - 1):
        a_sum = _sc_gather_sum(msg, a2b_flat)
        dmat = _sc_diff(a_sum, msg, b2a, b2revb)
        msg = _tc_update(dmat, inp, W_h)

    a_sum = _sc_gather_sum(msg, a2b_flat)
    Wo_a = W_o[:ATOM_FDIM]
    Wo_h = W_o[ATOM_FDIM:]
    return _tc_final(f_atoms, a_sum, Wo_a, Wo_h, b_o.reshape(1, H))


# speedup vs baseline: 1.7057x; 1.6379x over previous
import functools

import jax
import jax.numpy as jnp
from jax import lax
from jax.experimental import pallas as pl
from jax.experimental.pallas import tpu as pltpu
from jax.experimental.pallas import tpu_sc as plsc

E = 320000
NA = 10000
MAX_NB = 32
BOND_FDIM = 144
ATOM_FDIM = 128
H = 128
DEPTH = 5
N_MOLS = 200
MOL_SIZE = 50

NC = 2
NS = 16
NW = NC * NS

A_BLK = 4
IDX_BLK = A_BLK * MAX_NB
GS_NBLK = NA // A_BLK
GS_KMAX = (GS_NBLK + NW - 1) // NW


def _sc_gather_sum_body(msg_hbm, a2b_hbm, out_hbm,
                        idx0, idx1, rows0, rows1, outv0, outv1,
                        semg0, semg1, sems0, sems1):
    wid = lax.axis_index("s") * NC + lax.axis_index("c")
    bufs = ((idx0, rows0, outv0, semg0, sems0),
            (idx1, rows1, outv1, semg1, sems1))

    def fire(blk, idx, rows, semg):
        pltpu.sync_copy(a2b_hbm.at[pl.ds(blk * IDX_BLK, IDX_BLK)], idx)
        pltpu.make_async_copy(msg_hbm.at[idx], rows, semg).start()

    fire(wid, idx0, rows0, semg0)

    def pair(k2, _):
        for b in range(2):
            idx, rows, outv, semg, sems = bufs[b]
            nidx, nrows, _, nsemg, _ = bufs[1 - b]
            k = k2 * 2 + b
            blk = k * NW + wid

            @pl.when(blk + NW < GS_NBLK)
            def _():
                fire(blk + NW, nidx, nrows, nsemg)

            @pl.when(blk < GS_NBLK)
            def _():
                pltpu.make_async_copy(msg_hbm.at[idx], rows, semg).wait()

                @pl.when(k >= 2)
                def _():
                    pltpu.make_async_copy(
                        outv, out_hbm.at[pl.ds(0, A_BLK)], sems).wait()

                for a in range(A_BLK):
                    for c in range(H // 16):
                        acc = rows[a * MAX_NB, pl.ds(c * 16, 16)]
                        for r in range(1, MAX_NB):
                            acc = acc + rows[a * MAX_NB + r, pl.ds(c * 16, 16)]
                        outv[a, pl.ds(c * 16, 16)] = acc
                pltpu.make_async_copy(
                    outv, out_hbm.at[pl.ds(blk * A_BLK, A_BLK)], sems).start()

        return _

    lax.fori_loop(0, (GS_KMAX + 1) // 2, pair, None)
    pltpu.make_async_copy(outv0, out_hbm.at[pl.ds(0, A_BLK)], sems0).wait()
    pltpu.make_async_copy(outv1, out_hbm.at[pl.ds(0, A_BLK)], sems1).wait()


_sc_gather_sum = functools.partial(
    pl.kernel,
    out_type=jax.ShapeDtypeStruct((NA, H), jnp.float32),
    mesh=plsc.VectorSubcoreMesh(core_axis_name="c", subcore_axis_name="s"),
    scratch_types=[
        pltpu.VMEM((IDX_BLK,), jnp.int32),
        pltpu.VMEM((IDX_BLK,), jnp.int32),
        pltpu.VMEM((IDX_BLK, H), jnp.float32),
        pltpu.VMEM((IDX_BLK, H), jnp.float32),
        pltpu.VMEM((A_BLK, H), jnp.float32),
        pltpu.VMEM((A_BLK, H), jnp.float32),
        pltpu.SemaphoreType.DMA,
        pltpu.SemaphoreType.DMA,
        pltpu.SemaphoreType.DMA,
        pltpu.SemaphoreType.DMA,
    ],
)(_sc_gather_sum_body)


B_BLK = 128
DF_NBLK = E // B_BLK
DF_KMAX = (DF_NBLK + NW - 1) // NW


def _sc_diff_body(asum_hbm, msg_hbm, b2a_hbm, b2revb_hbm, out_hbm,
                  ia0, ia1, ir0, ir1, ga0, ga1, gm0, gm1,
                  semg0, semg1, sems0, sems1):
    wid = lax.axis_index("s") * NC + lax.axis_index("c")
    bufs = ((ia0, ir0, ga0, gm0, semg0, sems0),
            (ia1, ir1, ga1, gm1, semg1, sems1))

    def fire(blk, ia, ir, ga, gm, semg):
        base = blk * B_BLK
        pltpu.sync_copy(b2a_hbm.at[pl.ds(base, B_BLK)], ia)
        pltpu.sync_copy(b2revb_hbm.at[pl.ds(base, B_BLK)], ir)
        pltpu.make_async_copy(asum_hbm.at[ia], ga, semg).start()
        pltpu.make_async_copy(msg_hbm.at[ir], gm, semg).start()

    fire(wid, ia0, ir0, ga0, gm0, semg0)

    def pair(k2, _):
        for b in range(2):
            ia, ir, ga, gm, semg, sems = bufs[b]
            nia, nir, nga, ngm, nsemg, nsems = bufs[1 - b]
            k = k2 * 2 + b
            blk = k * NW + wid

            @pl.when(blk + NW < DF_NBLK)
            def _():
                @pl.when(k >= 1)
                def _():
                    pltpu.make_async_copy(
                        nga, out_hbm.at[pl.ds(0, B_BLK)], nsems).wait()

                fire(blk + NW, nia, nir, nga, ngm, nsemg)

            @pl.when(blk < DF_NBLK)
            def _():
                pltpu.make_async_copy(asum_hbm.at[ia], ga, semg).wait()
                pltpu.make_async_copy(msg_hbm.at[ir], gm, semg).wait()

                def comp(r8, _):
                    for rr in range(8):
                        r = r8 * 8 + rr
                        for c in range(H // 16):
                            ga[r, pl.ds(c * 16, 16)] = (
                                ga[r, pl.ds(c * 16, 16)]
                                - gm[r, pl.ds(c * 16, 16)]
                            )
                    return _

                lax.fori_loop(0, B_BLK // 8, comp, None)
                pltpu.make_async_copy(
                    ga, out_hbm.at[pl.ds(blk * B_BLK, B_BLK)], sems).start()

        return _

    lax.fori_loop(0, (DF_KMAX + 1) // 2, pair, None)
    pltpu.make_async_copy(ga0, out_hbm.at[pl.ds(0, B_BLK)], sems0).wait()
    pltpu.make_async_copy(ga1, out_hbm.at[pl.ds(0, B_BLK)], sems1).wait()


_sc_diff = functools.partial(
    pl.kernel,
    out_type=jax.ShapeDtypeStruct((E, H), jnp.float32),
    mesh=plsc.VectorSubcoreMesh(core_axis_name="c", subcore_axis_name="s"),
    scratch_types=[
        pltpu.VMEM((B_BLK,), jnp.int32),
        pltpu.VMEM((B_BLK,), jnp.int32),
        pltpu.VMEM((B_BLK,), jnp.int32),
        pltpu.VMEM((B_BLK,), jnp.int32),
        pltpu.VMEM((B_BLK, H), jnp.float32),
        pltpu.VMEM((B_BLK, H), jnp.float32),
        pltpu.VMEM((B_BLK, H), jnp.float32),
        pltpu.VMEM((B_BLK, H), jnp.float32),
        pltpu.SemaphoreType.DMA,
        pltpu.SemaphoreType.DMA,
        pltpu.SemaphoreType.DMA,
        pltpu.SemaphoreType.DMA,
    ],
)(_sc_diff_body)


TC_R = 6400


def _tc_input_body(fb_ref, wi_ref, inp_ref, msg_ref):
    x = jnp.dot(fb_ref[...], wi_ref[...], preferred_element_type=jnp.float32)
    inp_ref[...] = x
    msg_ref[...] = jnp.maximum(x, 0.0)


def _tc_input(f_bonds, W_i):
    grid = (E // TC_R,)
    return pl.pallas_call(
        _tc_input_body,
        grid=grid,
        in_specs=[
            pl.BlockSpec((TC_R, BOND_FDIM), lambda i: (i, 0)),
            pl.BlockSpec((BOND_FDIM, H), lambda i: (0, 0)),
        ],
        out_specs=[
            pl.BlockSpec((TC_R, H), lambda i: (i, 0)),
            pl.BlockSpec((TC_R, H), lambda i: (i, 0)),
        ],
        out_shape=[
            jax.ShapeDtypeStruct((E, H), jnp.float32),
            jax.ShapeDtypeStruct((E, H), jnp.float32),
        ],
    )(f_bonds, W_i)


def _tc_update_body(d_ref, inp_ref, wh_ref, out_ref):
    x = jnp.dot(d_ref[...], wh_ref[...], preferred_element_type=jnp.float32)
    out_ref[...] = jnp.maximum(inp_ref[...] + x, 0.0)


def _tc_update(dmat, inp, W_h):
    grid = (E // TC_R,)
    return pl.pallas_call(
        _tc_update_body,
        grid=grid,
        in_specs=[
            pl.BlockSpec((TC_R, H), lambda i: (i, 0)),
            pl.BlockSpec((TC_R, H), lambda i: (i, 0)),
            pl.BlockSpec((H, H), lambda i: (0, 0)),
        ],
        out_specs=pl.BlockSpec((TC_R, H), lambda i: (i, 0)),
        out_shape=jax.ShapeDtypeStruct((E, H), jnp.float32),
    )(dmat, inp, W_h)


FIN_R = 2000
FIN_M = FIN_R // MOL_SIZE


def _tc_final_body(fa_ref, as_ref, woa_ref, woh_ref, bo_ref, out_ref):
    ah = jnp.dot(fa_ref[...], woa_ref[...], preferred_element_type=jnp.float32)
    ah = ah + jnp.dot(as_ref[...], woh_ref[...],
                      preferred_element_type=jnp.float32)
    ah = jnp.maximum(ah + bo_ref[...], 0.0)
    rows = lax.broadcasted_iota(jnp.int32, (FIN_M, FIN_R), 0)
    cols = lax.broadcasted_iota(jnp.int32, (FIN_M, FIN_R), 1)
    pool = jnp.where(cols // MOL_SIZE == rows, 1.0 / MOL_SIZE, 0.0)
    out_ref[...] = jnp.dot(pool, ah, preferred_element_type=jnp.float32)


def _tc_final(f_atoms, a_sum, Wo_a, Wo_h, b_o2):
    grid = (NA // FIN_R,)
    return pl.pallas_call(
        _tc_final_body,
        grid=grid,
        in_specs=[
            pl.BlockSpec((FIN_R, ATOM_FDIM), lambda i: (i, 0)),
            pl.BlockSpec((FIN_R, H), lambda i: (i, 0)),
            pl.BlockSpec((ATOM_FDIM, H), lambda i: (0, 0)),
            pl.BlockSpec((H, H), lambda i: (0, 0)),
            pl.BlockSpec((1, H), lambda i: (0, 0)),
        ],
        out_specs=pl.BlockSpec((FIN_M, H), lambda i: (i, 0)),
        out_shape=jax.ShapeDtypeStruct((N_MOLS, H), jnp.float32),
    )(f_atoms, a_sum, Wo_a, Wo_h, b_o2)


def kernel(f_atoms, f_bonds, a2b, b2a, b2revb, W_i, W_h, W_o, b_o):
    a2b_flat = a2b.reshape(NA * MAX_NB).astype(jnp.int32)
    b2a = b2a.astype(jnp.int32)
    b2revb = b2revb.astype(jnp.int32)

    inp, msg = _tc_input(f_bonds, W_i)
    for _ in range(DEPTH - 1):
        a_sum = _sc_gather_sum(msg, a2b_flat)
        dmat = _sc_diff(a_sum, msg, b2a, b2revb)
        msg = _tc_update(dmat, inp, W_h)

    a_sum = _sc_gather_sum(msg, a2b_flat)
    Wo_a = W_o[:ATOM_FDIM]
    Wo_h = W_o[ATOM_FDIM:]
    return _tc_final(f_atoms, a_sum, Wo_a, Wo_h, b_o.reshape(1, H))
